# Initial kernel scaffold; baseline (speedup 1.0000x reference)
import jax
import jax.numpy as jnp
from jax.experimental import pallas as pl


def kernel(x, edge_index, W_self, W_neigh, b_neigh):
    raise NotImplementedError("write your pallas kernel here")


# speedup vs baseline: 5.0189x; 5.0189x over previous
import functools

import jax
import jax.numpy as jnp
from jax import lax
from jax.experimental import pallas as pl
from jax.experimental.pallas import tpu as pltpu
from jax.experimental.pallas import tpu_sc as plsc

N = 10000
E = 320000
D = 128
DH = D // 2

NC = 2
NS = 16
B = 128
NB = E // B
NIT = (NB + NS - 1) // NS
NPAD = 10112
SL = NPAD // NS
DW = 8

_CHUNKS = [(0, 128), (128, 128), (256, 128), (384, 128), (512, 120)]

_sc_mesh = plsc.VectorSubcoreMesh(core_axis_name="c", subcore_axis_name="s")


@functools.partial(
    pl.kernel,
    out_type=[
        jax.ShapeDtypeStruct((NC, NPAD, DH), jnp.float32),
        jax.ShapeDtypeStruct((NPAD, DW), jnp.float32),
    ],
    mesh=_sc_mesh,
    compiler_params=pltpu.CompilerParams(use_tc_tiling_on_sc=False),
    scratch_types=[
        pltpu.VMEM((B,), jnp.int32),
        pltpu.VMEM((B,), jnp.int32),
        pltpu.VMEM((B, DH), jnp.float32),
        pltpu.VMEM((B, DW), jnp.float32),
        pltpu.VMEM((B, DW), jnp.float32),
        pltpu.VMEM_SHARED((NPAD, DH), jnp.float32),
        pltpu.VMEM_SHARED((NPAD, DW), jnp.float32),
        pltpu.SemaphoreType.DMA,
    ],
)
def _sc_aggregate(xlo_hbm, xhi_hbm, src_hbm, dst_hbm, zrow_hbm, zdeg_hbm,
                  ones_hbm,
                  agg_out, deg_out,
                  sidx, didx, rows, ones_v, degst, agg_sh, deg_sh, sem):
    c = lax.axis_index("c")
    s = lax.axis_index("s")

    pltpu.sync_copy(zrow_hbm, rows)
    pltpu.sync_copy(zdeg_hbm, degst)
    for off, sz in _CHUNKS:
        pltpu.sync_copy(rows.at[pl.ds(0, sz)],
                        agg_sh.at[pl.ds(s * SL + off, sz)])
    pltpu.sync_copy(ones_hbm, ones_v)

    @pl.when(c == 1)
    def _():
        for off, sz in _CHUNKS:
            pltpu.sync_copy(degst.at[pl.ds(0, sz)],
                            deg_sh.at[pl.ds(s * SL + off, sz)])

    plsc.subcore_barrier()

    nit = lax.select(s < NB - (NIT - 1) * NS, NIT, NIT - 1)

    def make_body(x_hbm, with_deg):
        def body(i, carry):
            off = (s + i * NS) * B
            pltpu.sync_copy(src_hbm.at[pl.ds(off, B)], sidx)
            pltpu.sync_copy(dst_hbm.at[pl.ds(off, B)], didx)
            pltpu.async_copy(x_hbm.at[sidx], rows, sem).wait()
            pltpu.sync_copy(rows, agg_sh.at[didx], add=True)
            if with_deg:
                pltpu.sync_copy(ones_v, deg_sh.at[didx], add=True)
            return carry
        return body

    @pl.when(c == 0)
    def _():
        lax.fori_loop(0, nit, make_body(xlo_hbm, False), 0)

    @pl.when(c == 1)
    def _():
        lax.fori_loop(0, nit, make_body(xhi_hbm, True), 0)

    plsc.subcore_barrier()

    for off, sz in _CHUNKS:
        pltpu.sync_copy(agg_sh.at[pl.ds(s * SL + off, sz)], rows.at[pl.ds(0, sz)])
        pltpu.sync_copy(rows.at[pl.ds(0, sz)],
                        agg_out.at[c, pl.ds(s * SL + off, sz)])

    @pl.when(c == 1)
    def _():
        for off, sz in _CHUNKS:
            pltpu.sync_copy(deg_sh.at[pl.ds(s * SL + off, sz)],
                            degst.at[pl.ds(0, sz)])
            pltpu.sync_copy(degst.at[pl.ds(0, sz)],
                            deg_out.at[pl.ds(s * SL + off, sz)])


BLK = 1000


def _tc_combine_body(x_ref, alo_ref, ahi_ref, d_ref,
                     ws_ref, wn_lo_ref, wn_hi_ref, b_ref, o_ref):
    recip = 1.0 / jnp.maximum(d_ref[:, 0:1], 1.0)
    o_ref[...] = (
        jnp.dot(x_ref[...], ws_ref[...], preferred_element_type=jnp.float32)
        + jnp.dot(alo_ref[...] * recip, wn_lo_ref[...],
                  preferred_element_type=jnp.float32)
        + jnp.dot(ahi_ref[...] * recip, wn_hi_ref[...],
                  preferred_element_type=jnp.float32)
        + b_ref[...]
    )


def _tc_combine(x, a_lo, a_hi, degf, W_self, Wn_lo, Wn_hi, b2d):
    grid = (N // BLK,)
    row_spec = pl.BlockSpec((BLK, D), lambda i: (i, 0))
    half_spec = pl.BlockSpec((BLK, DH), lambda i: (i, 0))
    deg_spec = pl.BlockSpec((BLK, DW), lambda i: (i, 0))
    w_spec = pl.BlockSpec((D, D), lambda i: (0, 0))
    wh_spec = pl.BlockSpec((DH, D), lambda i: (0, 0))
    b_spec = pl.BlockSpec((1, D), lambda i: (0, 0))
    return pl.pallas_call(
        _tc_combine_body,
        grid=grid,
        in_specs=[row_spec, half_spec, half_spec, deg_spec,
                  w_spec, wh_spec, wh_spec, b_spec],
        out_specs=row_spec,
        out_shape=jax.ShapeDtypeStruct((N, D), jnp.float32),
    )(x, a_lo, a_hi, degf, W_self, Wn_lo, Wn_hi, b2d)


def kernel(x, edge_index, W_self, W_neigh, b_neigh):
    src = edge_index[0]
    dst = edge_index[1]
    x_lo = x[:, :DH]
    x_hi = x[:, DH:]
    zrow = jnp.zeros((B, DH), jnp.float32)
    zdeg = jnp.zeros((B, DW), jnp.float32)
    ones = jnp.ones((B, DW), jnp.float32)
    agg, deg = _sc_aggregate(x_lo, x_hi, src, dst, zrow, zdeg, ones)
    out = _tc_combine(
        x,
        agg[0, :N], agg[1, :N],
        deg[:N],
        W_self, W_neigh[:DH], W_neigh[DH:],
        b_neigh.reshape(1, D),
    )
    return out
